# baseline (speedup 1.0000x reference)
import functools

import jax
import jax.numpy as jnp
from jax import lax
from jax.experimental import pallas as pl
from jax.experimental.pallas import tpu as pltpu
from jax.experimental.pallas import tpu_sc as plsc

N = 100000
E = 1600000
NG = 512
DH = 32
HALF = 16
CH = 128
E_CHUNKS = E // CH
N_CHUNKS = N // CH
N_TAIL = N - N_CHUNKS * CH
ROWCH = 400
NRC = N // ROWCH
NT = 16
NSC = 2
NW = NT * NSC

_mesh = functools.partial(
    plsc.VectorSubcoreMesh, core_axis_name="c", subcore_axis_name="s")
_sc_params = pltpu.CompilerParams(use_tc_tiling_on_sc=False)


def _tile_row_loop(s, fn):
    count = (NRC - 1 - s) // NT + 1

    def body(i, _):
        fn((s + i * NT) * ROWCH)
        return 0

    lax.fori_loop(0, count, body, 0)


def _sc_degree(dst, zeros_n, ones_c):
    out_t = jax.ShapeDtypeStruct((N,), jnp.float32)

    @functools.partial(
        pl.kernel,
        out_type=(out_t, out_t),
        mesh=_mesh(),
        compiler_params=_sc_params,
        scratch_types=[
            pltpu.VMEM((CH,), jnp.int32),
            pltpu.VMEM((CH,), jnp.float32),
            pltpu.VMEM((ROWCH,), jnp.float32),
            pltpu.VMEM_SHARED((N,), jnp.float32),
            pltpu.SemaphoreType.DMA,
        ],
    )
    def k(dst_hbm, zn_hbm, ones_hbm, o0_hbm, o1_hbm, idx_v, ones_v, z_v, acc, sem):
        c = lax.axis_index("c")
        s = lax.axis_index("s")
        w = s * NSC + c
        pltpu.sync_copy(ones_hbm, ones_v)
        pltpu.sync_copy(zn_hbm.at[pl.ds(0, ROWCH)], z_v)
        _tile_row_loop(
            s, lambda off: pltpu.sync_copy(z_v, acc.at[pl.ds(off, ROWCH)]))
        plsc.subcore_barrier()

        count = (E_CHUNKS - 1 - w) // NW + 1

        def body(i, _):
            j = w + i * NW
            pltpu.sync_copy(dst_hbm.at[pl.ds(j * CH, CH)], idx_v)
            pltpu.sync_copy(ones_v, acc.at[idx_v], add=True)
            return 0

        lax.fori_loop(0, count, body, 0)
        plsc.subcore_barrier()

        def wb(o_hbm):
            def cp(off):
                pltpu.sync_copy(acc.at[pl.ds(off, ROWCH)], z_v)
                pltpu.sync_copy(z_v, o_hbm.at[pl.ds(off, ROWCH)])
            _tile_row_loop(s, cp)

        @pl.when(c == 0)
        def _():
            wb(o0_hbm)

        @pl.when(c == 1)
        def _():
            wb(o1_hbm)

    return k(dst, zeros_n, ones_c)


IB = 8
N_SUPER = E_CHUNKS // IB
E_TAIL_CHUNKS = E_CHUNKS - N_SUPER * IB


def _sc_edge_pass(g0, g1, eidx2):
    out_t = jax.ShapeDtypeStruct((N, HALF), jnp.float32)

    @functools.partial(
        pl.kernel,
        out_type=(out_t, out_t),
        mesh=_mesh(),
        compiler_params=_sc_params,
        scratch_types=[
            pltpu.VMEM((2 * IB, CH), jnp.int32),
            [pltpu.VMEM((CH, HALF), jnp.float32) for _ in range(IB)],
            pltpu.VMEM((ROWCH, HALF), jnp.float32),
            pltpu.VMEM_SHARED((N, HALF), jnp.float32),
            pltpu.SemaphoreType.DMA,
            pltpu.SemaphoreType.DMA,
        ],
    )
    def k(g0_hbm, g1_hbm, eidx_hbm, o0_hbm, o1_hbm,
          eidx, rows, stage, acc, sem, ssem):
        c = lax.axis_index("c")
        s = lax.axis_index("s")

        def init(g_hbm):
            def cp(off):
                pltpu.sync_copy(g_hbm.at[pl.ds(off, ROWCH)], stage)
                pltpu.sync_copy(stage, acc.at[pl.ds(off, ROWCH)])
            _tile_row_loop(s, cp)

        @pl.when(c == 0)
        def _():
            init(g0_hbm)

        @pl.when(c == 1)
        def _():
            init(g1_hbm)

        plsc.subcore_barrier()

        count = (N_SUPER - 1 - s) // NT + 1

        def loop(g_hbm):
            def super_block(e0, nb):
                pltpu.sync_copy(eidx_hbm.at[pl.ds(2 * e0, 2 * nb)],
                                eidx.at[pl.ds(0, 2 * nb)])
                hs = [pltpu.async_copy(g_hbm.at[eidx.at[2 * b]], rows[b], sem)
                      for b in range(nb)]
                for h in hs:
                    h.wait()
                for b in range(nb):
                    pltpu.sync_copy(rows[b], acc.at[eidx.at[2 * b + 1]], add=True)

            def body(i, _):
                super_block((s + i * NT) * IB, IB)
                return 0
            lax.fori_loop(0, count, body, 0)

            @pl.when(s == 0)
            def _():
                super_block(N_SUPER * IB, E_TAIL_CHUNKS)

        @pl.when(c == 0)
        def _():
            loop(g0_hbm)

        @pl.when(c == 1)
        def _():
            loop(g1_hbm)

        plsc.subcore_barrier()

        def wb(o_hbm):
            def cp(off):
                pltpu.sync_copy(acc.at[pl.ds(off, ROWCH)], stage)
                pltpu.sync_copy(stage, o_hbm.at[pl.ds(off, ROWCH)])
            _tile_row_loop(s, cp)

        @pl.when(c == 0)
        def _():
            wb(o0_hbm)

        @pl.when(c == 1)
        def _():
            wb(o1_hbm)

    return k(g0, g1, eidx2)


def _sc_pool(h3, batch, zeros_g, zeros_ng, ones_c):
    @functools.partial(
        pl.kernel,
        out_type=(jax.ShapeDtypeStruct((NG, DH), jnp.float32),
                  jax.ShapeDtypeStruct((NG, DH), jnp.float32),
                  jax.ShapeDtypeStruct((NG,), jnp.float32),
                  jax.ShapeDtypeStruct((NG,), jnp.float32)),
        mesh=_mesh(),
        compiler_params=_sc_params,
        scratch_types=[
            pltpu.VMEM((CH,), jnp.int32),
            pltpu.VMEM((CH, DH), jnp.float32),
            pltpu.VMEM((CH,), jnp.float32),
            pltpu.VMEM((N_TAIL,), jnp.int32),
            pltpu.VMEM((N_TAIL, DH), jnp.float32),
            pltpu.VMEM((N_TAIL,), jnp.float32),
            pltpu.VMEM((NG, DH), jnp.float32),
            pltpu.VMEM((NG,), jnp.float32),
            pltpu.VMEM_SHARED((NG, DH), jnp.float32),
            pltpu.VMEM_SHARED((NG,), jnp.float32),
            pltpu.SemaphoreType.DMA,
        ],
    )
    def k(h_hbm, b_hbm, zg_hbm, zng_hbm, ones_hbm,
          os0_hbm, os1_hbm, oc0_hbm, oc1_hbm,
          bidx, rows, ones_v, bidx_t, rows_t, ones_t, zg_v, zng_v,
          accs, accc, sem):
        c = lax.axis_index("c")
        s = lax.axis_index("s")
        w = s * NSC + c
        pltpu.sync_copy(ones_hbm, ones_v)
        pltpu.sync_copy(ones_hbm.at[pl.ds(0, N_TAIL)], ones_t)

        @pl.when(s == 0)
        def _():
            pltpu.sync_copy(zg_hbm, zg_v)
            pltpu.sync_copy(zng_hbm, zng_v)
            pltpu.sync_copy(zg_v, accs)
            pltpu.sync_copy(zng_v, accc)

        plsc.subcore_barrier()

        count = (N_CHUNKS - 1 - w) // NW + 1

        def body(i, _):
            j = w + i * NW
            pltpu.sync_copy(b_hbm.at[pl.ds(j * CH, CH)], bidx)
            pltpu.sync_copy(h_hbm.at[pl.ds(j * CH, CH)], rows)
            pltpu.sync_copy(rows, accs.at[bidx], add=True)
            pltpu.sync_copy(ones_v, accc.at[bidx], add=True)
            return 0

        lax.fori_loop(0, count, body, 0)

        @pl.when(w == 0)
        def _():
            t0 = N_CHUNKS * CH
            pltpu.sync_copy(b_hbm.at[pl.ds(t0, N_TAIL)], bidx_t)
            pltpu.sync_copy(h_hbm.at[pl.ds(t0, N_TAIL)], rows_t)
            pltpu.sync_copy(rows_t, accs.at[bidx_t], add=True)
            pltpu.sync_copy(ones_t, accc.at[bidx_t], add=True)

        plsc.subcore_barrier()

        @pl.when(s == 0)
        def _():
            pltpu.sync_copy(accs, zg_v)
            pltpu.sync_copy(accc, zng_v)

        @pl.when((s == 0) & (c == 0))
        def _():
            pltpu.sync_copy(zg_v, os0_hbm)
            pltpu.sync_copy(zng_v, oc0_hbm)

        @pl.when((s == 0) & (c == 1))
        def _():
            pltpu.sync_copy(zg_v, os1_hbm)
            pltpu.sync_copy(zng_v, oc1_hbm)

    return k(h3, batch, zeros_g, zeros_ng, ones_c)


_TCR = 2000


def _tc_pre(x, p0, p1, W1):
    def body(x_ref, p0_ref, p1_ref, w_ref, dv_ref, g0_ref, g1_ref):
        deg = p0_ref[...] + p1_ref[...] + 1.0
        dv = lax.rsqrt(deg)
        h = jnp.dot(x_ref[...], w_ref[...], preferred_element_type=jnp.float32)
        g = h * dv
        dv_ref[...] = dv
        g0_ref[...] = g[:, :HALF]
        g1_ref[...] = g[:, HALF:]

    return pl.pallas_call(
        body,
        grid=(N // _TCR,),
        in_specs=[
            pl.BlockSpec((_TCR, 6), lambda i: (i, 0)),
            pl.BlockSpec((_TCR, 1), lambda i: (i, 0)),
            pl.BlockSpec((_TCR, 1), lambda i: (i, 0)),
            pl.BlockSpec((6, DH), lambda i: (0, 0)),
        ],
        out_specs=[
            pl.BlockSpec((_TCR, 1), lambda i: (i, 0)),
            pl.BlockSpec((_TCR, HALF), lambda i: (i, 0)),
            pl.BlockSpec((_TCR, HALF), lambda i: (i, 0)),
        ],
        out_shape=[
            jax.ShapeDtypeStruct((N, 1), jnp.float32),
            jax.ShapeDtypeStruct((N, HALF), jnp.float32),
            jax.ShapeDtypeStruct((N, HALF), jnp.float32),
        ],
    )(x, p0, p1, W1)


def _tc_mid(s0, s1, dv, W, bias):
    def body(s0_ref, s1_ref, dv_ref, w_ref, b_ref, g0_ref, g1_ref):
        dvb = dv_ref[...]
        h = jnp.concatenate([s0_ref[...], s1_ref[...]], axis=1) * dvb + b_ref[...]
        h = jnp.maximum(h, 0.0)
        g = jnp.dot(h, w_ref[...], preferred_element_type=jnp.float32) * dvb
        g0_ref[...] = g[:, :HALF]
        g1_ref[...] = g[:, HALF:]

    return pl.pallas_call(
        body,
        grid=(N // _TCR,),
        in_specs=[
            pl.BlockSpec((_TCR, HALF), lambda i: (i, 0)),
            pl.BlockSpec((_TCR, HALF), lambda i: (i, 0)),
            pl.BlockSpec((_TCR, 1), lambda i: (i, 0)),
            pl.BlockSpec((DH, DH), lambda i: (0, 0)),
            pl.BlockSpec((1, DH), lambda i: (0, 0)),
        ],
        out_specs=[
            pl.BlockSpec((_TCR, HALF), lambda i: (i, 0)),
            pl.BlockSpec((_TCR, HALF), lambda i: (i, 0)),
        ],
        out_shape=[
            jax.ShapeDtypeStruct((N, HALF), jnp.float32),
            jax.ShapeDtypeStruct((N, HALF), jnp.float32),
        ],
    )(s0, s1, dv, W, bias)


def _tc_fin(s0, s1, dv, bias):
    def body(s0_ref, s1_ref, dv_ref, b_ref, h_ref):
        h_ref[...] = (jnp.concatenate([s0_ref[...], s1_ref[...]], axis=1)
                      * dv_ref[...] + b_ref[...])

    return pl.pallas_call(
        body,
        grid=(N // _TCR,),
        in_specs=[
            pl.BlockSpec((_TCR, HALF), lambda i: (i, 0)),
            pl.BlockSpec((_TCR, HALF), lambda i: (i, 0)),
            pl.BlockSpec((_TCR, 1), lambda i: (i, 0)),
            pl.BlockSpec((1, DH), lambda i: (0, 0)),
        ],
        out_specs=pl.BlockSpec((_TCR, DH), lambda i: (i, 0)),
        out_shape=jax.ShapeDtypeStruct((N, DH), jnp.float32),
    )(s0, s1, dv, bias)


def kernel(x, edge_index, batch, W1, b1, W2, b2, W3, b3):
    src = edge_index[0]
    dst = edge_index[1]
    src2d = src.reshape(E_CHUNKS, CH)
    dst2d = dst.reshape(E_CHUNKS, CH)
    eidx2 = jnp.stack([src2d, dst2d], axis=1).reshape(2 * E_CHUNKS, CH)

    zeros_n = jnp.zeros((N,), jnp.float32)
    ones_c = jnp.ones((CH,), jnp.float32)
    zeros_g = jnp.zeros((NG, DH), jnp.float32)
    zeros_ng = jnp.zeros((NG,), jnp.float32)

    p0, p1 = _sc_degree(dst, zeros_n, ones_c)
    dv, g0, g1 = _tc_pre(x, p0[:, None], p1[:, None], W1)

    s0, s1 = _sc_edge_pass(g0, g1, eidx2)
    g0, g1 = _tc_mid(s0, s1, dv, W2, b1[None, :])
    s0, s1 = _sc_edge_pass(g0, g1, eidx2)
    g0, g1 = _tc_mid(s0, s1, dv, W3, b2[None, :])
    s0, s1 = _sc_edge_pass(g0, g1, eidx2)
    h3 = _tc_fin(s0, s1, dv, b3[None, :])

    sm0, sm1, ct0, ct1 = _sc_pool(h3, batch, zeros_g, zeros_ng, ones_c)
    tot = sm0 + sm1
    cnt = ct0 + ct1
    return tot / jnp.maximum(cnt, 1.0)[:, None]


# speedup vs baseline: 1.0880x; 1.0880x over previous
import functools

import jax
import jax.numpy as jnp
from jax import lax
from jax.experimental import pallas as pl
from jax.experimental.pallas import tpu as pltpu
from jax.experimental.pallas import tpu_sc as plsc

N = 100000
E = 1600000
NG = 512
DH = 32
HALF = 16
CH = 128
E_CHUNKS = E // CH
N_CHUNKS = N // CH
N_TAIL = N - N_CHUNKS * CH
ROWCH = 400
NRC = N // ROWCH
NT = 16
NSC = 2
NW = NT * NSC

_mesh = functools.partial(
    plsc.VectorSubcoreMesh, core_axis_name="c", subcore_axis_name="s")
_sc_params = pltpu.CompilerParams(use_tc_tiling_on_sc=False)


def _tile_row_loop(s, fn):
    count = (NRC - 1 - s) // NT + 1

    def body(i, _):
        fn((s + i * NT) * ROWCH)
        return 0

    lax.fori_loop(0, count, body, 0)


def _sc_degree(dst, zeros_n, ones_c):
    out_t = jax.ShapeDtypeStruct((N,), jnp.float32)

    @functools.partial(
        pl.kernel,
        out_type=(out_t, out_t),
        mesh=_mesh(),
        compiler_params=_sc_params,
        scratch_types=[
            pltpu.VMEM((CH,), jnp.int32),
            pltpu.VMEM((CH,), jnp.float32),
            pltpu.VMEM((ROWCH,), jnp.float32),
            pltpu.VMEM_SHARED((N,), jnp.float32),
            pltpu.SemaphoreType.DMA,
        ],
    )
    def k(dst_hbm, zn_hbm, ones_hbm, o0_hbm, o1_hbm, idx_v, ones_v, z_v, acc, sem):
        c = lax.axis_index("c")
        s = lax.axis_index("s")
        w = s * NSC + c
        pltpu.sync_copy(ones_hbm, ones_v)
        pltpu.sync_copy(zn_hbm.at[pl.ds(0, ROWCH)], z_v)
        _tile_row_loop(
            s, lambda off: pltpu.sync_copy(z_v, acc.at[pl.ds(off, ROWCH)]))
        plsc.subcore_barrier()

        count = (E_CHUNKS - 1 - w) // NW + 1

        def body(i, _):
            j = w + i * NW
            pltpu.sync_copy(dst_hbm.at[pl.ds(j * CH, CH)], idx_v)
            pltpu.sync_copy(ones_v, acc.at[idx_v], add=True)
            return 0

        lax.fori_loop(0, count, body, 0)
        plsc.subcore_barrier()

        def wb(o_hbm):
            def cp(off):
                pltpu.sync_copy(acc.at[pl.ds(off, ROWCH)], z_v)
                pltpu.sync_copy(z_v, o_hbm.at[pl.ds(off, ROWCH)])
            _tile_row_loop(s, cp)

        @pl.when(c == 0)
        def _():
            wb(o0_hbm)

        @pl.when(c == 1)
        def _():
            wb(o1_hbm)

    return k(dst, zeros_n, ones_c)


IB = 10
N_SUPER = E_CHUNKS // IB


def _sc_edge_pass(g0, g1, esup):
    out_t = jax.ShapeDtypeStruct((N, HALF), jnp.float32)

    @functools.partial(
        pl.kernel,
        out_type=(out_t, out_t),
        mesh=_mesh(),
        compiler_params=_sc_params,
        scratch_types=[
            pltpu.VMEM((2, IB * CH), jnp.int32),
            pltpu.VMEM((IB * CH, HALF), jnp.float32),
            pltpu.VMEM((ROWCH, HALF), jnp.float32),
            pltpu.VMEM_SHARED((N, HALF), jnp.float32),
            pltpu.SemaphoreType.DMA,
        ],
    )
    def k(g0_hbm, g1_hbm, esup_hbm, o0_hbm, o1_hbm,
          ebuf, rows3, stage, acc, sem):
        c = lax.axis_index("c")
        s = lax.axis_index("s")

        def init(g_hbm):
            def cp(off):
                pltpu.sync_copy(g_hbm.at[pl.ds(off, ROWCH)], stage)
                pltpu.sync_copy(stage, acc.at[pl.ds(off, ROWCH)])
            _tile_row_loop(s, cp)

        @pl.when(c == 0)
        def _():
            init(g0_hbm)

        @pl.when(c == 1)
        def _():
            init(g1_hbm)

        plsc.subcore_barrier()

        count = (N_SUPER - 1 - s) // NT + 1

        def loop(g_hbm):
            def body(i, _):
                u = s + i * NT
                pltpu.sync_copy(esup_hbm.at[u], ebuf)
                pltpu.async_copy(g_hbm.at[ebuf.at[0]], rows3, sem).wait()
                pltpu.sync_copy(rows3, acc.at[ebuf.at[1]], add=True)
                return 0
            lax.fori_loop(0, count, body, 0)

        @pl.when(c == 0)
        def _():
            loop(g0_hbm)

        @pl.when(c == 1)
        def _():
            loop(g1_hbm)

        plsc.subcore_barrier()

        def wb(o_hbm):
            def cp(off):
                pltpu.sync_copy(acc.at[pl.ds(off, ROWCH)], stage)
                pltpu.sync_copy(stage, o_hbm.at[pl.ds(off, ROWCH)])
            _tile_row_loop(s, cp)

        @pl.when(c == 0)
        def _():
            wb(o0_hbm)

        @pl.when(c == 1)
        def _():
            wb(o1_hbm)

    return k(g0, g1, esup)


def _sc_pool(h3, batch, zeros_g, zeros_ng, ones_c):
    @functools.partial(
        pl.kernel,
        out_type=(jax.ShapeDtypeStruct((NG, DH), jnp.float32),
                  jax.ShapeDtypeStruct((NG, DH), jnp.float32),
                  jax.ShapeDtypeStruct((NG,), jnp.float32),
                  jax.ShapeDtypeStruct((NG,), jnp.float32)),
        mesh=_mesh(),
        compiler_params=_sc_params,
        scratch_types=[
            pltpu.VMEM((CH,), jnp.int32),
            pltpu.VMEM((CH, DH), jnp.float32),
            pltpu.VMEM((CH,), jnp.float32),
            pltpu.VMEM((N_TAIL,), jnp.int32),
            pltpu.VMEM((N_TAIL, DH), jnp.float32),
            pltpu.VMEM((N_TAIL,), jnp.float32),
            pltpu.VMEM((NG, DH), jnp.float32),
            pltpu.VMEM((NG,), jnp.float32),
            pltpu.VMEM_SHARED((NG, DH), jnp.float32),
            pltpu.VMEM_SHARED((NG,), jnp.float32),
            pltpu.SemaphoreType.DMA,
        ],
    )
    def k(h_hbm, b_hbm, zg_hbm, zng_hbm, ones_hbm,
          os0_hbm, os1_hbm, oc0_hbm, oc1_hbm,
          bidx, rows, ones_v, bidx_t, rows_t, ones_t, zg_v, zng_v,
          accs, accc, sem):
        c = lax.axis_index("c")
        s = lax.axis_index("s")
        w = s * NSC + c
        pltpu.sync_copy(ones_hbm, ones_v)
        pltpu.sync_copy(ones_hbm.at[pl.ds(0, N_TAIL)], ones_t)

        @pl.when(s == 0)
        def _():
            pltpu.sync_copy(zg_hbm, zg_v)
            pltpu.sync_copy(zng_hbm, zng_v)
            pltpu.sync_copy(zg_v, accs)
            pltpu.sync_copy(zng_v, accc)

        plsc.subcore_barrier()

        count = (N_CHUNKS - 1 - w) // NW + 1

        def body(i, _):
            j = w + i * NW
            pltpu.sync_copy(b_hbm.at[pl.ds(j * CH, CH)], bidx)
            pltpu.sync_copy(h_hbm.at[pl.ds(j * CH, CH)], rows)
            pltpu.sync_copy(rows, accs.at[bidx], add=True)
            pltpu.sync_copy(ones_v, accc.at[bidx], add=True)
            return 0

        lax.fori_loop(0, count, body, 0)

        @pl.when(w == 0)
        def _():
            t0 = N_CHUNKS * CH
            pltpu.sync_copy(b_hbm.at[pl.ds(t0, N_TAIL)], bidx_t)
            pltpu.sync_copy(h_hbm.at[pl.ds(t0, N_TAIL)], rows_t)
            pltpu.sync_copy(rows_t, accs.at[bidx_t], add=True)
            pltpu.sync_copy(ones_t, accc.at[bidx_t], add=True)

        plsc.subcore_barrier()

        @pl.when(s == 0)
        def _():
            pltpu.sync_copy(accs, zg_v)
            pltpu.sync_copy(accc, zng_v)

        @pl.when((s == 0) & (c == 0))
        def _():
            pltpu.sync_copy(zg_v, os0_hbm)
            pltpu.sync_copy(zng_v, oc0_hbm)

        @pl.when((s == 0) & (c == 1))
        def _():
            pltpu.sync_copy(zg_v, os1_hbm)
            pltpu.sync_copy(zng_v, oc1_hbm)

    return k(h3, batch, zeros_g, zeros_ng, ones_c)


_TCR = 2000


def _tc_pre(x, p0, p1, W1):
    def body(x_ref, p0_ref, p1_ref, w_ref, dv_ref, g0_ref, g1_ref):
        deg = p0_ref[...] + p1_ref[...] + 1.0
        dv = lax.rsqrt(deg)
        h = jnp.dot(x_ref[...], w_ref[...], preferred_element_type=jnp.float32)
        g = h * dv
        dv_ref[...] = dv
        g0_ref[...] = g[:, :HALF]
        g1_ref[...] = g[:, HALF:]

    return pl.pallas_call(
        body,
        grid=(N // _TCR,),
        in_specs=[
            pl.BlockSpec((_TCR, 6), lambda i: (i, 0)),
            pl.BlockSpec((_TCR, 1), lambda i: (i, 0)),
            pl.BlockSpec((_TCR, 1), lambda i: (i, 0)),
            pl.BlockSpec((6, DH), lambda i: (0, 0)),
        ],
        out_specs=[
            pl.BlockSpec((_TCR, 1), lambda i: (i, 0)),
            pl.BlockSpec((_TCR, HALF), lambda i: (i, 0)),
            pl.BlockSpec((_TCR, HALF), lambda i: (i, 0)),
        ],
        out_shape=[
            jax.ShapeDtypeStruct((N, 1), jnp.float32),
            jax.ShapeDtypeStruct((N, HALF), jnp.float32),
            jax.ShapeDtypeStruct((N, HALF), jnp.float32),
        ],
    )(x, p0, p1, W1)


def _tc_mid(s0, s1, dv, W, bias):
    def body(s0_ref, s1_ref, dv_ref, w_ref, b_ref, g0_ref, g1_ref):
        dvb = dv_ref[...]
        h = jnp.concatenate([s0_ref[...], s1_ref[...]], axis=1) * dvb + b_ref[...]
        h = jnp.maximum(h, 0.0)
        g = jnp.dot(h, w_ref[...], preferred_element_type=jnp.float32) * dvb
        g0_ref[...] = g[:, :HALF]
        g1_ref[...] = g[:, HALF:]

    return pl.pallas_call(
        body,
        grid=(N // _TCR,),
        in_specs=[
            pl.BlockSpec((_TCR, HALF), lambda i: (i, 0)),
            pl.BlockSpec((_TCR, HALF), lambda i: (i, 0)),
            pl.BlockSpec((_TCR, 1), lambda i: (i, 0)),
            pl.BlockSpec((DH, DH), lambda i: (0, 0)),
            pl.BlockSpec((1, DH), lambda i: (0, 0)),
        ],
        out_specs=[
            pl.BlockSpec((_TCR, HALF), lambda i: (i, 0)),
            pl.BlockSpec((_TCR, HALF), lambda i: (i, 0)),
        ],
        out_shape=[
            jax.ShapeDtypeStruct((N, HALF), jnp.float32),
            jax.ShapeDtypeStruct((N, HALF), jnp.float32),
        ],
    )(s0, s1, dv, W, bias)


def _tc_fin(s0, s1, dv, bias):
    def body(s0_ref, s1_ref, dv_ref, b_ref, h_ref):
        h_ref[...] = (jnp.concatenate([s0_ref[...], s1_ref[...]], axis=1)
                      * dv_ref[...] + b_ref[...])

    return pl.pallas_call(
        body,
        grid=(N // _TCR,),
        in_specs=[
            pl.BlockSpec((_TCR, HALF), lambda i: (i, 0)),
            pl.BlockSpec((_TCR, HALF), lambda i: (i, 0)),
            pl.BlockSpec((_TCR, 1), lambda i: (i, 0)),
            pl.BlockSpec((1, DH), lambda i: (0, 0)),
        ],
        out_specs=pl.BlockSpec((_TCR, DH), lambda i: (i, 0)),
        out_shape=jax.ShapeDtypeStruct((N, DH), jnp.float32),
    )(s0, s1, dv, bias)


def kernel(x, edge_index, batch, W1, b1, W2, b2, W3, b3):
    src = edge_index[0]
    dst = edge_index[1]
    src2d = src.reshape(E_CHUNKS, CH)
    dst2d = dst.reshape(E_CHUNKS, CH)
    esup = jnp.stack([src2d.reshape(N_SUPER, IB * CH),
                      dst2d.reshape(N_SUPER, IB * CH)], axis=1)

    zeros_n = jnp.zeros((N,), jnp.float32)
    ones_c = jnp.ones((CH,), jnp.float32)
    zeros_g = jnp.zeros((NG, DH), jnp.float32)
    zeros_ng = jnp.zeros((NG,), jnp.float32)

    p0, p1 = _sc_degree(dst, zeros_n, ones_c)
    dv, g0, g1 = _tc_pre(x, p0[:, None], p1[:, None], W1)

    s0, s1 = _sc_edge_pass(g0, g1, esup)
    g0, g1 = _tc_mid(s0, s1, dv, W2, b1[None, :])
    s0, s1 = _sc_edge_pass(g0, g1, esup)
    g0, g1 = _tc_mid(s0, s1, dv, W3, b2[None, :])
    s0, s1 = _sc_edge_pass(g0, g1, esup)
    h3 = _tc_fin(s0, s1, dv, b3[None, :])

    sm0, sm1, ct0, ct1 = _sc_pool(h3, batch, zeros_g, zeros_ng, ones_c)
    tot = sm0 + sm1
    cnt = ct0 + ct1
    return tot / jnp.maximum(cnt, 1.0)[:, None]


# speedup vs baseline: 1.1848x; 1.0890x over previous
import functools

import jax
import jax.numpy as jnp
from jax import lax
from jax.experimental import pallas as pl
from jax.experimental.pallas import tpu as pltpu
from jax.experimental.pallas import tpu_sc as plsc

N = 100000
E = 1600000
NG = 512
DH = 32
HALF = 16
CH = 128
E_CHUNKS = E // CH
N_CHUNKS = N // CH
N_TAIL = N - N_CHUNKS * CH
ROWCH = 400
NRC = N // ROWCH
NT = 16
NSC = 2
NW = NT * NSC

_mesh = functools.partial(
    plsc.VectorSubcoreMesh, core_axis_name="c", subcore_axis_name="s")
_sc_params = pltpu.CompilerParams(use_tc_tiling_on_sc=False)


def _tile_row_loop(s, fn):
    count = (NRC - 1 - s) // NT + 1

    def body(i, _):
        fn((s + i * NT) * ROWCH)
        return 0

    lax.fori_loop(0, count, body, 0)


def _sc_degree(dst, zeros_n, ones_c):
    out_t = jax.ShapeDtypeStruct((N,), jnp.float32)

    @functools.partial(
        pl.kernel,
        out_type=(out_t, out_t),
        mesh=_mesh(),
        compiler_params=_sc_params,
        scratch_types=[
            pltpu.VMEM((IB * CH,), jnp.int32),
            pltpu.VMEM((IB * CH,), jnp.float32),
            pltpu.VMEM((ROWCH,), jnp.float32),
            pltpu.VMEM_SHARED((N,), jnp.float32),
            pltpu.SemaphoreType.DMA,
        ],
    )
    def k(dst_hbm, zn_hbm, ones_hbm, o0_hbm, o1_hbm, idx_v, ones_v, z_v, acc, sem):
        c = lax.axis_index("c")
        s = lax.axis_index("s")
        w = s * NSC + c
        pltpu.sync_copy(ones_hbm, ones_v)
        pltpu.sync_copy(zn_hbm.at[pl.ds(0, ROWCH)], z_v)
        _tile_row_loop(
            s, lambda off: pltpu.sync_copy(z_v, acc.at[pl.ds(off, ROWCH)]))
        plsc.subcore_barrier()

        count = (N_SUPER - 1 - w) // NW + 1

        def body(i, _):
            u = w + i * NW
            pltpu.sync_copy(dst_hbm.at[u], idx_v)
            pltpu.sync_copy(ones_v, acc.at[idx_v], add=True)
            return 0

        lax.fori_loop(0, count, body, 0)
        plsc.subcore_barrier()

        def wb(o_hbm):
            def cp(off):
                pltpu.sync_copy(acc.at[pl.ds(off, ROWCH)], z_v)
                pltpu.sync_copy(z_v, o_hbm.at[pl.ds(off, ROWCH)])
            _tile_row_loop(s, cp)

        @pl.when(c == 0)
        def _():
            wb(o0_hbm)

        @pl.when(c == 1)
        def _():
            wb(o1_hbm)

    return k(dst, zeros_n, ones_c)


IB = 10
N_SUPER = E_CHUNKS // IB


def _sc_edge_pass(g0, g1, esup):
    out_t = jax.ShapeDtypeStruct((N, HALF), jnp.float32)

    @functools.partial(
        pl.kernel,
        out_type=(out_t, out_t),
        mesh=_mesh(),
        compiler_params=_sc_params,
        scratch_types=[
            pltpu.VMEM((2, IB * CH), jnp.int32),
            pltpu.VMEM((IB * CH, HALF), jnp.float32),
            pltpu.VMEM((ROWCH, HALF), jnp.float32),
            pltpu.VMEM_SHARED((N, HALF), jnp.float32),
            pltpu.SemaphoreType.DMA,
        ],
    )
    def k(g0_hbm, g1_hbm, esup_hbm, o0_hbm, o1_hbm,
          ebuf, rows3, stage, acc, sem):
        c = lax.axis_index("c")
        s = lax.axis_index("s")

        def init(g_hbm):
            def cp(off):
                pltpu.sync_copy(g_hbm.at[pl.ds(off, ROWCH)], stage)
                pltpu.sync_copy(stage, acc.at[pl.ds(off, ROWCH)])
            _tile_row_loop(s, cp)

        @pl.when(c == 0)
        def _():
            init(g0_hbm)

        @pl.when(c == 1)
        def _():
            init(g1_hbm)

        plsc.subcore_barrier()

        count = (N_SUPER - 1 - s) // NT + 1

        def loop(g_hbm):
            def body(i, _):
                u = s + i * NT
                pltpu.sync_copy(esup_hbm.at[u], ebuf)
                pltpu.async_copy(g_hbm.at[ebuf.at[0]], rows3, sem).wait()
                pltpu.sync_copy(rows3, acc.at[ebuf.at[1]], add=True)
                return 0
            lax.fori_loop(0, count, body, 0)

        @pl.when(c == 0)
        def _():
            loop(g0_hbm)

        @pl.when(c == 1)
        def _():
            loop(g1_hbm)

        plsc.subcore_barrier()

        def wb(o_hbm):
            def cp(off):
                pltpu.sync_copy(acc.at[pl.ds(off, ROWCH)], stage)
                pltpu.sync_copy(stage, o_hbm.at[pl.ds(off, ROWCH)])
            _tile_row_loop(s, cp)

        @pl.when(c == 0)
        def _():
            wb(o0_hbm)

        @pl.when(c == 1)
        def _():
            wb(o1_hbm)

    return k(g0, g1, esup)


def _sc_pool(h3, batch, zeros_g, zeros_ng, ones_c):
    @functools.partial(
        pl.kernel,
        out_type=(jax.ShapeDtypeStruct((NG, DH), jnp.float32),
                  jax.ShapeDtypeStruct((NG, DH), jnp.float32),
                  jax.ShapeDtypeStruct((NG,), jnp.float32),
                  jax.ShapeDtypeStruct((NG,), jnp.float32)),
        mesh=_mesh(),
        compiler_params=_sc_params,
        scratch_types=[
            pltpu.VMEM((CH,), jnp.int32),
            pltpu.VMEM((CH, DH), jnp.float32),
            pltpu.VMEM((CH,), jnp.float32),
            pltpu.VMEM((N_TAIL,), jnp.int32),
            pltpu.VMEM((N_TAIL, DH), jnp.float32),
            pltpu.VMEM((N_TAIL,), jnp.float32),
            pltpu.VMEM((NG, DH), jnp.float32),
            pltpu.VMEM((NG,), jnp.float32),
            pltpu.VMEM_SHARED((NG, DH), jnp.float32),
            pltpu.VMEM_SHARED((NG,), jnp.float32),
            pltpu.SemaphoreType.DMA,
        ],
    )
    def k(h_hbm, b_hbm, zg_hbm, zng_hbm, ones_hbm,
          os0_hbm, os1_hbm, oc0_hbm, oc1_hbm,
          bidx, rows, ones_v, bidx_t, rows_t, ones_t, zg_v, zng_v,
          accs, accc, sem):
        c = lax.axis_index("c")
        s = lax.axis_index("s")
        w = s * NSC + c
        pltpu.sync_copy(ones_hbm, ones_v)
        pltpu.sync_copy(ones_hbm.at[pl.ds(0, N_TAIL)], ones_t)

        @pl.when(s == 0)
        def _():
            pltpu.sync_copy(zg_hbm, zg_v)
            pltpu.sync_copy(zng_hbm, zng_v)
            pltpu.sync_copy(zg_v, accs)
            pltpu.sync_copy(zng_v, accc)

        plsc.subcore_barrier()

        count = (N_CHUNKS - 1 - w) // NW + 1

        def body(i, _):
            j = w + i * NW
            pltpu.sync_copy(b_hbm.at[pl.ds(j * CH, CH)], bidx)
            pltpu.sync_copy(h_hbm.at[pl.ds(j * CH, CH)], rows)
            pltpu.sync_copy(rows, accs.at[bidx], add=True)
            pltpu.sync_copy(ones_v, accc.at[bidx], add=True)
            return 0

        lax.fori_loop(0, count, body, 0)

        @pl.when(w == 0)
        def _():
            t0 = N_CHUNKS * CH
            pltpu.sync_copy(b_hbm.at[pl.ds(t0, N_TAIL)], bidx_t)
            pltpu.sync_copy(h_hbm.at[pl.ds(t0, N_TAIL)], rows_t)
            pltpu.sync_copy(rows_t, accs.at[bidx_t], add=True)
            pltpu.sync_copy(ones_t, accc.at[bidx_t], add=True)

        plsc.subcore_barrier()

        @pl.when(s == 0)
        def _():
            pltpu.sync_copy(accs, zg_v)
            pltpu.sync_copy(accc, zng_v)

        @pl.when((s == 0) & (c == 0))
        def _():
            pltpu.sync_copy(zg_v, os0_hbm)
            pltpu.sync_copy(zng_v, oc0_hbm)

        @pl.when((s == 0) & (c == 1))
        def _():
            pltpu.sync_copy(zg_v, os1_hbm)
            pltpu.sync_copy(zng_v, oc1_hbm)

    return k(h3, batch, zeros_g, zeros_ng, ones_c)


_TCR = 2000


def _tc_pre(x, p0, p1, W1):
    def body(x_ref, p0_ref, p1_ref, w_ref, dv_ref, g0_ref, g1_ref):
        deg = p0_ref[...] + p1_ref[...] + 1.0
        dv = lax.rsqrt(deg)
        h = jnp.dot(x_ref[...], w_ref[...], preferred_element_type=jnp.float32)
        g = h * dv
        dv_ref[...] = dv
        g0_ref[...] = g[:, :HALF]
        g1_ref[...] = g[:, HALF:]

    return pl.pallas_call(
        body,
        grid=(N // _TCR,),
        in_specs=[
            pl.BlockSpec((_TCR, 6), lambda i: (i, 0)),
            pl.BlockSpec((_TCR, 1), lambda i: (i, 0)),
            pl.BlockSpec((_TCR, 1), lambda i: (i, 0)),
            pl.BlockSpec((6, DH), lambda i: (0, 0)),
        ],
        out_specs=[
            pl.BlockSpec((_TCR, 1), lambda i: (i, 0)),
            pl.BlockSpec((_TCR, HALF), lambda i: (i, 0)),
            pl.BlockSpec((_TCR, HALF), lambda i: (i, 0)),
        ],
        out_shape=[
            jax.ShapeDtypeStruct((N, 1), jnp.float32),
            jax.ShapeDtypeStruct((N, HALF), jnp.float32),
            jax.ShapeDtypeStruct((N, HALF), jnp.float32),
        ],
    )(x, p0, p1, W1)


def _tc_mid(s0, s1, dv, W, bias):
    def body(s0_ref, s1_ref, dv_ref, w_ref, b_ref, g0_ref, g1_ref):
        dvb = dv_ref[...]
        h = jnp.concatenate([s0_ref[...], s1_ref[...]], axis=1) * dvb + b_ref[...]
        h = jnp.maximum(h, 0.0)
        g = jnp.dot(h, w_ref[...], preferred_element_type=jnp.float32) * dvb
        g0_ref[...] = g[:, :HALF]
        g1_ref[...] = g[:, HALF:]

    return pl.pallas_call(
        body,
        grid=(N // _TCR,),
        in_specs=[
            pl.BlockSpec((_TCR, HALF), lambda i: (i, 0)),
            pl.BlockSpec((_TCR, HALF), lambda i: (i, 0)),
            pl.BlockSpec((_TCR, 1), lambda i: (i, 0)),
            pl.BlockSpec((DH, DH), lambda i: (0, 0)),
            pl.BlockSpec((1, DH), lambda i: (0, 0)),
        ],
        out_specs=[
            pl.BlockSpec((_TCR, HALF), lambda i: (i, 0)),
            pl.BlockSpec((_TCR, HALF), lambda i: (i, 0)),
        ],
        out_shape=[
            jax.ShapeDtypeStruct((N, HALF), jnp.float32),
            jax.ShapeDtypeStruct((N, HALF), jnp.float32),
        ],
    )(s0, s1, dv, W, bias)


def _tc_fin(s0, s1, dv, bias):
    def body(s0_ref, s1_ref, dv_ref, b_ref, h_ref):
        h_ref[...] = (jnp.concatenate([s0_ref[...], s1_ref[...]], axis=1)
                      * dv_ref[...] + b_ref[...])

    return pl.pallas_call(
        body,
        grid=(N // _TCR,),
        in_specs=[
            pl.BlockSpec((_TCR, HALF), lambda i: (i, 0)),
            pl.BlockSpec((_TCR, HALF), lambda i: (i, 0)),
            pl.BlockSpec((_TCR, 1), lambda i: (i, 0)),
            pl.BlockSpec((1, DH), lambda i: (0, 0)),
        ],
        out_specs=pl.BlockSpec((_TCR, DH), lambda i: (i, 0)),
        out_shape=jax.ShapeDtypeStruct((N, DH), jnp.float32),
    )(s0, s1, dv, bias)


def kernel(x, edge_index, batch, W1, b1, W2, b2, W3, b3):
    src = edge_index[0]
    dst = edge_index[1]
    src2d = src.reshape(E_CHUNKS, CH)
    dst2d = dst.reshape(E_CHUNKS, CH)
    esup = jnp.stack([src2d.reshape(N_SUPER, IB * CH),
                      dst2d.reshape(N_SUPER, IB * CH)], axis=1)

    zeros_n = jnp.zeros((N,), jnp.float32)
    ones_c = jnp.ones((CH,), jnp.float32)
    ones_big = jnp.ones((IB * CH,), jnp.float32)
    zeros_g = jnp.zeros((NG, DH), jnp.float32)
    zeros_ng = jnp.zeros((NG,), jnp.float32)

    p0, p1 = _sc_degree(dst.reshape(N_SUPER, IB * CH), zeros_n, ones_big)
    dv, g0, g1 = _tc_pre(x, p0[:, None], p1[:, None], W1)

    s0, s1 = _sc_edge_pass(g0, g1, esup)
    g0, g1 = _tc_mid(s0, s1, dv, W2, b1[None, :])
    s0, s1 = _sc_edge_pass(g0, g1, esup)
    g0, g1 = _tc_mid(s0, s1, dv, W3, b2[None, :])
    s0, s1 = _sc_edge_pass(g0, g1, esup)
    h3 = _tc_fin(s0, s1, dv, b3[None, :])

    sm0, sm1, ct0, ct1 = _sc_pool(h3, batch, zeros_g, zeros_ng, ones_c)
    tot = sm0 + sm1
    cnt = ct0 + ct1
    return tot / jnp.maximum(cnt, 1.0)[:, None]
